# baseline (speedup 1.0000x reference)
import functools

import jax
import jax.numpy as jnp
from jax import lax
from jax.experimental import pallas as pl
from jax.experimental.pallas import tpu as pltpu
from jax.experimental.pallas import tpu_sc as plsc

VOCAB = 100000
EMB_D = 32
WIN = 2
BATCH = 1024

_NIDX = BATCH * WIN
_NW = 32
_PER_W = _NIDX // _NW


def _sc_gather(table, idx):
    mesh = plsc.VectorSubcoreMesh(core_axis_name="c", subcore_axis_name="s")

    @functools.partial(
        pl.kernel,
        out_type=jax.ShapeDtypeStruct((_NIDX, EMB_D), jnp.float32),
        mesh=mesh,
        compiler_params=pltpu.CompilerParams(use_tc_tiling_on_sc=False),
        scratch_types=[
            pltpu.VMEM((_PER_W,), jnp.int32),
            pltpu.VMEM((_PER_W, EMB_D), jnp.float32),
            pltpu.SemaphoreType.DMA,
        ],
    )
    def k(table_hbm, idx_hbm, out_hbm, idx_v, rows_v, sem):
        wid = lax.axis_index("s") * 2 + lax.axis_index("c")
        base = wid * _PER_W
        pltpu.sync_copy(idx_hbm.at[pl.ds(base, _PER_W)], idx_v)
        pltpu.async_copy(table_hbm.at[idx_v], rows_v, sem).wait()
        pltpu.sync_copy(rows_v, out_hbm.at[pl.ds(base, _PER_W)])

    return k(table, idx)


_VBLK = 4096
_NSTEP = pl.cdiv(VOCAB, _VBLK)


def _matmul_body(wt_ref, flat_ref, b_ref, out_ref):
    prod = lax.dot_general(
        wt_ref[...], flat_ref[...],
        (((0,), (1,)), ((), ())),
        preferred_element_type=jnp.float32,
    )
    out_ref[...] = prod + b_ref[...]


def _tc_matmul_t(Wt, flat, bcol):
    return pl.pallas_call(
        _matmul_body,
        grid=(_NSTEP,),
        in_specs=[
            pl.BlockSpec((WIN * EMB_D, _VBLK), lambda i: (0, i)),
            pl.BlockSpec((BATCH, WIN * EMB_D), lambda i: (0, 0)),
            pl.BlockSpec((_VBLK, 1), lambda i: (i, 0)),
        ],
        out_specs=pl.BlockSpec((_VBLK, BATCH), lambda i: (i, 0)),
        out_shape=jax.ShapeDtypeStruct((VOCAB, BATCH), jnp.float32),
        compiler_params=pltpu.CompilerParams(
            dimension_semantics=("parallel",),
            vmem_limit_bytes=128 * 1024 * 1024,
        ),
    )(Wt, flat, bcol)


def kernel(x, emb, W, b):
    idx = x.reshape(-1).astype(jnp.int32)
    flat = _sc_gather(emb, idx).reshape(BATCH, WIN * EMB_D)
    out_t = _tc_matmul_t(W.T, flat, b.reshape(VOCAB, 1))
    return out_t.T
